# Initial kernel scaffold; baseline (speedup 1.0000x reference)
import functools

import jax
import jax.numpy as jnp
from jax import lax
from jax.experimental import pallas as pl
from jax.experimental.pallas import tpu as pltpu
from jax.experimental.pallas import tpu_sc as plsc

NC = 2
NS = 16
NW = NC * NS
K = 128


def _sc_kernel(x_hbm, src_hbm, dst_hbm, g_hbm, z2_hbm, z1_hbm,
               part_hbm, gsum_hbm,
               src_v, dst_v, g_v, rows_v, gsum_v, acc, sem_g, sem_s,
               *, ch, n_pad, d):
    cid = lax.axis_index("c")
    sid = lax.axis_index("s")
    wid = sid * NC + cid
    rpt = n_pad // NS

    pltpu.sync_copy(src_hbm.at[wid], src_v)
    pltpu.sync_copy(dst_hbm.at[wid], dst_v)
    pltpu.sync_copy(g_hbm.at[wid], g_v)

    pltpu.sync_copy(z1_hbm, gsum_v)
    for b in range(rpt // K):
        pltpu.sync_copy(z2_hbm, acc.at[pl.ds(sid * rpt + b * K, K)])
    plsc.subcore_barrier()

    def chunk_body(c, _):
        pltpu.async_copy(x_hbm.at[src_v.at[c]], rows_v, sem_g).wait()

        def row_body(r, _):
            gb = plsc.load_gather(
                g_v, [jnp.full((16,), c, jnp.int32),
                      jnp.full((16,), r, jnp.int32)])
            for j in range(d // 16):
                sl = pl.ds(j * 16, 16)
                rows_v[r, sl] = rows_v[r, sl] * gb
            return 0
        lax.fori_loop(0, K, row_body, 0)

        for j in range(K // 16):
            sl = pl.ds(j * 16, 16)
            plsc.addupdate_scatter(gsum_v, [dst_v[c, sl]], g_v[c, sl])

        pltpu.async_copy(rows_v, acc.at[dst_v.at[c]], sem_s, add=True).wait()
        return 0

    lax.fori_loop(0, ch, chunk_body, 0)
    plsc.subcore_barrier()

    for b in range(rpt // K):
        sl = pl.ds(sid * rpt + b * K, K)
        pltpu.sync_copy(acc.at[sl], part_hbm.at[cid, sl])
    pltpu.sync_copy(gsum_v, gsum_hbm.at[wid])


def _combine_kernel(p_ref, gs_ref, x_ref, o_ref):
    gs = jnp.sum(gs_ref[...], axis=0)
    o_ref[...] = p_ref[0] + p_ref[1] - (1.0 + gs)[:, None] * x_ref[...]


@jax.jit
def kernel(x, g, edge_index):
    n, d = x.shape
    e = g.shape[0]
    dst = edge_index[0]
    src = edge_index[1]

    ch = -(-e // (NW * K))
    e_pad = NW * K * ch
    n_pad = -(-n // (NS * K)) * (NS * K)

    pad = e_pad - e
    src_p = jnp.concatenate([src, jnp.zeros((pad,), jnp.int32)]).reshape(NW, ch, K)
    dst_p = jnp.concatenate([dst, jnp.zeros((pad,), jnp.int32)]).reshape(NW, ch, K)
    g_p = jnp.concatenate([g, jnp.zeros((pad,), jnp.float32)]).reshape(NW, ch, K)
    x_p = jnp.pad(x, ((0, n_pad - n), (0, 0)))
    z2 = jnp.zeros((K, d), jnp.float32)
    z1 = jnp.zeros((n_pad,), jnp.float32)

    mesh = plsc.VectorSubcoreMesh(core_axis_name="c", subcore_axis_name="s")
    part, gsum = pl.kernel(
        functools.partial(_sc_kernel, ch=ch, n_pad=n_pad, d=d),
        out_type=(jax.ShapeDtypeStruct((NC, n_pad, d), jnp.float32),
                  jax.ShapeDtypeStruct((NW, n_pad), jnp.float32)),
        mesh=mesh,
        scratch_types=[
            pltpu.VMEM((ch, K), jnp.int32),
            pltpu.VMEM((ch, K), jnp.int32),
            pltpu.VMEM((ch, K), jnp.float32),
            pltpu.VMEM((K, d), jnp.float32),
            pltpu.VMEM((n_pad,), jnp.float32),
            pltpu.VMEM_SHARED((n_pad, d), jnp.float32),
            pltpu.SemaphoreType.DMA,
            pltpu.SemaphoreType.DMA,
        ],
    )(x_p, src_p, dst_p, g_p, z2, z1)

    out = pl.pallas_call(
        _combine_kernel,
        grid=(n_pad // K,),
        in_specs=[
            pl.BlockSpec((NC, K, d), lambda i: (0, i, 0)),
            pl.BlockSpec((NW, K), lambda i: (0, i)),
            pl.BlockSpec((K, d), lambda i: (i, 0)),
        ],
        out_specs=pl.BlockSpec((K, d), lambda i: (i, 0)),
        out_shape=jax.ShapeDtypeStruct((n_pad, d), jnp.float32),
    )(part, gsum, x_p)

    return out[:n]


# speedup vs baseline: 4.0843x; 4.0843x over previous
import functools

import jax
import jax.numpy as jnp
from jax import lax
from jax.experimental import pallas as pl
from jax.experimental.pallas import tpu as pltpu
from jax.experimental.pallas import tpu_sc as plsc

NC = 2
NS = 16
NW = NC * NS
K = 128


def _sc_kernel(x_hbm, idx_hbm, gg_hbm, z_hbm, z1_hbm,
               part_hbm, gpart_hbm,
               idx_v, gg_v, rows_v, acc, gacc,
               sem_g, sem_s, sem_q,
               *, ch, n_pad, d):
    cid = lax.axis_index("c")
    sid = lax.axis_index("s")
    wid = sid * NC + cid
    rpt = n_pad // NS

    pltpu.sync_copy(z_hbm, acc.at[pl.ds(sid * rpt, rpt)])
    pltpu.sync_copy(z1_hbm, gacc.at[pl.ds(sid * rpt, rpt)])
    plsc.subcore_barrier()

    def chunk_body(c, _):
        pltpu.sync_copy(idx_hbm.at[wid, c], idx_v)
        pltpu.sync_copy(gg_hbm.at[wid, c], gg_v)
        pltpu.async_copy(x_hbm.at[idx_v.at[0]], rows_v, sem_g).wait()

        def scale_body(jj, _):
            for rr in range(8):
                gb = gg_v[1 + jj, pl.ds(rr * 16, 16)]
                r = jj * 8 + rr
                for j in range(d // 16):
                    sl = pl.ds(j * 16, 16)
                    rows_v[r, sl] = rows_v[r, sl] * gb
            return 0
        lax.fori_loop(0, K // 8, scale_body, 0)

        q = pltpu.async_copy(gg_v.at[0], gacc.at[idx_v.at[1]], sem_q,
                             add=True)
        pltpu.async_copy(rows_v, acc.at[idx_v.at[1]], sem_s, add=True).wait()
        q.wait()
        return 0

    lax.fori_loop(0, ch, chunk_body, 0)
    plsc.subcore_barrier()

    sl = pl.ds(sid * rpt, rpt)
    pltpu.sync_copy(acc.at[sl], part_hbm.at[cid, sl])
    pltpu.sync_copy(gacc.at[sl],
                    gpart_hbm.at[pl.ds(cid * n_pad + sid * rpt, rpt)])


def _combine_kernel(p_ref, gp_ref, x_ref, o_ref):
    gs = gp_ref[0] + gp_ref[1]
    o_ref[...] = p_ref[0] + p_ref[1] - (1.0 + gs)[:, None] * x_ref[...]


@jax.jit
def kernel(x, g, edge_index):
    n, d = x.shape
    e = g.shape[0]
    dst = edge_index[0]
    src = edge_index[1]

    ch = -(-e // (NW * K))
    e_pad = NW * K * ch
    n_pad = -(-n // (NS * K)) * (NS * K)
    rpt = n_pad // NS

    pad = e_pad - e
    src_p = jnp.concatenate([src, jnp.zeros((pad,), jnp.int32)]).reshape(NW, ch, K)
    dst_p = jnp.concatenate([dst, jnp.zeros((pad,), jnp.int32)]).reshape(NW, ch, K)
    g_p = jnp.concatenate([g, jnp.zeros((pad,), jnp.float32)]).reshape(NW, ch, K)
    idx_p = jnp.stack([src_p, dst_p], axis=2)
    gbc = jnp.broadcast_to(g_p[..., None], (NW, ch, K, 16))
    gg_p = jnp.concatenate([g_p[:, :, None, :],
                            gbc.reshape(NW, ch, 16, K)], axis=2)
    x_p = jnp.pad(x, ((0, n_pad - n), (0, 0)))
    z = jnp.zeros((rpt, d), jnp.float32)
    z1 = jnp.zeros((rpt,), jnp.float32)

    mesh = plsc.VectorSubcoreMesh(core_axis_name="c", subcore_axis_name="s",
                                  num_cores=NC, num_subcores=NS)
    part, gpart = pl.kernel(
        functools.partial(_sc_kernel, ch=ch, n_pad=n_pad, d=d),
        out_type=(jax.ShapeDtypeStruct((NC, n_pad, d), jnp.float32),
                  jax.ShapeDtypeStruct((NC * n_pad,), jnp.float32)),
        mesh=mesh,
        scratch_types=[
            pltpu.VMEM((2, K), jnp.int32),
            pltpu.VMEM((17, K), jnp.float32),
            pltpu.VMEM((K, d), jnp.float32),
            pltpu.VMEM_SHARED((n_pad, d), jnp.float32),
            pltpu.VMEM_SHARED((n_pad,), jnp.float32),
            pltpu.SemaphoreType.DMA,
            pltpu.SemaphoreType.DMA,
            pltpu.SemaphoreType.DMA,
        ],
    )(x_p, idx_p, gg_p, z, z1)

    out = pl.pallas_call(
        _combine_kernel,
        grid=(n_pad // K,),
        in_specs=[
            pl.BlockSpec((NC, K, d), lambda i: (0, i, 0)),
            pl.BlockSpec((NC, K), lambda i: (0, i)),
            pl.BlockSpec((K, d), lambda i: (i, 0)),
        ],
        out_specs=pl.BlockSpec((K, d), lambda i: (i, 0)),
        out_shape=jax.ShapeDtypeStruct((n_pad, d), jnp.float32),
    )(part, gpart.reshape(NC, n_pad), x_p)

    return out[:n]


# speedup vs baseline: 5.2355x; 1.2818x over previous
import functools

import jax
import jax.numpy as jnp
from jax import lax
from jax.experimental import pallas as pl
from jax.experimental.pallas import tpu as pltpu
from jax.experimental.pallas import tpu_sc as plsc

NC = 2
NS = 16
NW = NC * NS
K = 128


def _sc_kernel(x_hbm, idx_hbm, gg_hbm, z_hbm, z1_hbm,
               part_hbm, gpart_hbm,
               idx_v, gg_v, rows_v, acc, gacc,
               sem_g, sem_s, sem_q,
               *, ch, n_pad, d):
    cid = lax.axis_index("c")
    sid = lax.axis_index("s")
    wid = sid * NC + cid
    rpt = n_pad // NS

    pltpu.sync_copy(z_hbm, acc.at[pl.ds(sid * rpt, rpt)])
    pltpu.sync_copy(z1_hbm, gacc.at[pl.ds(sid * rpt, rpt)])
    plsc.subcore_barrier()

    def stage(c, b):
        pltpu.sync_copy(idx_hbm.at[wid, c], idx_v.at[b])
        pltpu.sync_copy(gg_hbm.at[wid, c], gg_v.at[b])

    def gather_start(b):
        return pltpu.async_copy(x_hbm.at[idx_v.at[b, 0]], rows_v.at[b],
                                sem_g)

    def scatter_start(b):
        pltpu.async_copy(gg_v.at[b, 0], gacc.at[idx_v.at[b, 1]],
                         sem_q, add=True)
        pltpu.async_copy(rows_v.at[b], acc.at[idx_v.at[b, 1]],
                         sem_s, add=True)

    def scatter_wait(b):
        pltpu.make_async_copy(gg_v.at[b, 0], gacc.at[idx_v.at[b, 1]],
                              sem_q).wait()
        pltpu.make_async_copy(rows_v.at[b], acc.at[idx_v.at[b, 1]],
                              sem_s).wait()

    def gather_wait(b):
        pltpu.make_async_copy(x_hbm.at[idx_v.at[b, 0]], rows_v.at[b],
                              sem_g).wait()

    stage(0, 0)
    gather_start(0)

    def chunk_body(c, _):
        b = c % 2
        nb = 1 - b

        @pl.when(c + 1 < ch)
        def _():
            @pl.when(c >= 1)
            def _():
                scatter_wait(nb)
            stage(c + 1, nb)
            gather_start(nb)

        gather_wait(b)

        def scale_body(jj, _):
            for rr in range(8):
                gb = gg_v[b, 1 + jj, pl.ds(rr * 16, 16)]
                r = jj * 8 + rr
                for j in range(d // 16):
                    sl = pl.ds(j * 16, 16)
                    rows_v[b, r, sl] = rows_v[b, r, sl] * gb
            return 0
        lax.fori_loop(0, K // 8, scale_body, 0)

        scatter_start(b)
        return 0

    lax.fori_loop(0, ch, chunk_body, 0)
    scatter_wait((ch - 1) % 2)
    @pl.when(ch >= 2)
    def _():
        scatter_wait(ch % 2)
    plsc.subcore_barrier()

    sl = pl.ds(sid * rpt, rpt)
    pltpu.sync_copy(acc.at[sl], part_hbm.at[cid, sl])
    pltpu.sync_copy(gacc.at[sl],
                    gpart_hbm.at[pl.ds(cid * n_pad + sid * rpt, rpt)])


def _combine_kernel(p_ref, gp_ref, x_ref, o_ref):
    gs = gp_ref[0] + gp_ref[1]
    o_ref[...] = p_ref[0] + p_ref[1] - (1.0 + gs)[:, None] * x_ref[...]


@jax.jit
def kernel(x, g, edge_index):
    n, d = x.shape
    e = g.shape[0]
    dst = edge_index[0]
    src = edge_index[1]

    ch = -(-e // (NW * K))
    e_pad = NW * K * ch
    n_pad = -(-n // (NS * K)) * (NS * K)
    rpt = n_pad // NS

    pad = e_pad - e
    src_p = jnp.concatenate([src, jnp.zeros((pad,), jnp.int32)]).reshape(NW, ch, K)
    dst_p = jnp.concatenate([dst, jnp.zeros((pad,), jnp.int32)]).reshape(NW, ch, K)
    g_p = jnp.concatenate([g, jnp.zeros((pad,), jnp.float32)]).reshape(NW, ch, K)
    idx_p = jnp.stack([src_p, dst_p], axis=2)
    gbc = jnp.broadcast_to(g_p[..., None], (NW, ch, K, 16))
    gg_p = jnp.concatenate([g_p[:, :, None, :],
                            gbc.reshape(NW, ch, 16, K)], axis=2)
    x_p = jnp.pad(x, ((0, n_pad - n), (0, 0)))
    z = jnp.zeros((rpt, d), jnp.float32)
    z1 = jnp.zeros((rpt,), jnp.float32)

    mesh = plsc.VectorSubcoreMesh(core_axis_name="c", subcore_axis_name="s",
                                  num_cores=NC, num_subcores=NS)
    part, gpart = pl.kernel(
        functools.partial(_sc_kernel, ch=ch, n_pad=n_pad, d=d),
        out_type=(jax.ShapeDtypeStruct((NC, n_pad, d), jnp.float32),
                  jax.ShapeDtypeStruct((NC * n_pad,), jnp.float32)),
        mesh=mesh,
        scratch_types=[
            pltpu.VMEM((2, 2, K), jnp.int32),
            pltpu.VMEM((2, 17, K), jnp.float32),
            pltpu.VMEM((2, K, d), jnp.float32),
            pltpu.VMEM_SHARED((n_pad, d), jnp.float32),
            pltpu.VMEM_SHARED((n_pad,), jnp.float32),
            pltpu.SemaphoreType.DMA,
            pltpu.SemaphoreType.DMA,
            pltpu.SemaphoreType.DMA,
        ],
    )(x_p, idx_p, gg_p, z, z1)

    out = pl.pallas_call(
        _combine_kernel,
        grid=(n_pad // K,),
        in_specs=[
            pl.BlockSpec((NC, K, d), lambda i: (0, i, 0)),
            pl.BlockSpec((NC, K), lambda i: (0, i)),
            pl.BlockSpec((K, d), lambda i: (i, 0)),
        ],
        out_specs=pl.BlockSpec((K, d), lambda i: (i, 0)),
        out_shape=jax.ShapeDtypeStruct((n_pad, d), jnp.float32),
    )(part, gpart.reshape(NC, n_pad), x_p)

    return out[:n]
